# Initial kernel scaffold; baseline (speedup 1.0000x reference)
import jax
import jax.numpy as jnp
from jax.experimental import pallas as pl


def kernel(x, equi_index, atom_emb, chg_emb, ring_emb, mrs_emb, W1, b1, W2, b2, W3, b3):
    raise NotImplementedError("write your pallas kernel here")


# speedup vs baseline: 1.9853x; 1.9853x over previous
import jax
import jax.numpy as jnp
from jax import lax
from jax.experimental import pallas as pl
from jax.experimental.pallas import tpu as pltpu
from jax.experimental.pallas import tpu_sc as plsc

N_ATOM, N_CHG, N_RING = 100, 21, 8
RAW = 256
HID = 512
T1_ROWS = 144
NPAD = 102400
BLK = 512
GPAD = 25088
GARBAGE_GROUP = 25000
NCHUNK = 4
CCOLS = 128
NWIN = 4
WIN = 6272
ACC_ROWS = WIN + 8

_SC_MESH = dict(core_axis_name="c", subcore_axis_name="s")


def _prep_body(e_ref, w1_ref, out_ref):
    out_ref[...] = jnp.dot(e_ref[...], w1_ref[...],
                           preferred_element_type=jnp.float32)


def _make_t1(atom_emb, chg_emb, ring_emb, mrs_emb, W1):
    rows = []
    off = 0
    for tab in (atom_emb, chg_emb, ring_emb, mrs_emb):
        v, d = tab.shape
        rows.append(jnp.pad(tab, ((0, 0), (off, RAW - off - d))))
        off += d
    e = jnp.concatenate(rows, axis=0)
    e = jnp.pad(e, ((0, T1_ROWS - e.shape[0]), (0, 0)))
    return pl.pallas_call(
        _prep_body,
        out_shape=jax.ShapeDtypeStruct((T1_ROWS, HID), jnp.float32),
    )(e, W1)


def _gelu(v):
    return 0.5 * v * (1.0 + lax.erf(v * 0.7071067811865476))


def _mlp_body(jidx_ref, t1_ref, w2_ref, w3_ref, b1_ref, b2_ref, b3_ref,
              out_ref):
    j = jidx_ref[0]
    iota = lax.broadcasted_iota(jnp.int32, (BLK, T1_ROWS), 1)
    oh = ((iota == j[:, 0:1]).astype(jnp.float32)
          + (iota == j[:, 1:2]).astype(jnp.float32)
          + (iota == j[:, 2:3]).astype(jnp.float32)
          + (iota == j[:, 3:4]).astype(jnp.float32))
    h = jnp.dot(oh, t1_ref[...], preferred_element_type=jnp.float32)
    h = _gelu(h + b1_ref[0:1, :])
    h = jnp.dot(h, w2_ref[...], preferred_element_type=jnp.float32)
    h = _gelu(h + b2_ref[0:1, :])
    h = jnp.dot(h, w3_ref[...], preferred_element_type=jnp.float32)
    h = _gelu(h + b3_ref[0:1, :])
    out_ref[...] = h


def _run_mlp(jidx, t1, W2, W3, b1, b2, b3):
    nblk = NPAD // BLK
    bspec = pl.BlockSpec((8, HID), lambda i: (0, 0))
    return pl.pallas_call(
        _mlp_body,
        grid=(nblk,),
        in_specs=[
            pl.BlockSpec((1, BLK, 4), lambda i: (i, 0, 0)),
            pl.BlockSpec((T1_ROWS, HID), lambda i: (0, 0)),
            pl.BlockSpec((HID, HID), lambda i: (0, 0)),
            pl.BlockSpec((HID, HID), lambda i: (0, 0)),
            bspec, bspec, bspec,
        ],
        out_specs=pl.BlockSpec((BLK, HID), lambda i: (i, 0)),
        out_shape=jax.ShapeDtypeStruct((NPAD, HID), jnp.float32),
    )(jidx, t1, W2, W3, b1, b2, b3)


def _scatter_body(h3_hbm, seg_hbm, sums_hbm, cnt_hbm,
                  idx_v, idx2_v, buf_v, zero_v, ones_v, acc_sh):
    c = lax.axis_index("c")
    s = lax.axis_index("s")
    node_base = s * (NPAD // 16)
    zrow0 = s * (WIN // 16)

    def _z(i, _):
        for l in range(8):
            zero_v[i, pl.ds(l * 16, 16)] = jnp.zeros((16,), jnp.float32)
        return 0
    lax.fori_loop(0, 56, _z, 0)

    def _o(i, _):
        for l in range(8):
            ones_v[i, pl.ds(l * 16, 16)] = jnp.ones((16,), jnp.float32)
        return 0
    lax.fori_loop(0, 128, _o, 0)

    pltpu.sync_copy(seg_hbm.at[pl.ds(s * 56, 56)], idx_v)

    def _zero_acc():
        for z in range(7):
            pltpu.sync_copy(zero_v, acc_sh.at[pl.ds(zrow0 + z * 56, 56)])

    def _writeout(dst):
        pltpu.sync_copy(acc_sh.at[pl.ds(zrow0, WIN // 16)], dst)

    for w in range(NWIN):
        base = w * WIN

        def _xf(k, _):
            for l in range(8):
                v = idx_v[k, pl.ds(l * 16, 16)] - base
                ok = (v >= 0) & (v < WIN)
                idx2_v[k, pl.ds(l * 16, 16)] = jnp.where(ok, v, WIN)
            return 0
        lax.fori_loop(0, 50, _xf, 0)

        for q in range(2):
            _zero_acc()
            plsc.subcore_barrier()

            def _chunk(k, _):
                col0 = (c * 2 + q) * CCOLS
                pltpu.sync_copy(
                    h3_hbm.at[pl.ds(node_base + k * 128, 128),
                              pl.ds(col0, CCOLS)],
                    buf_v)
                pltpu.sync_copy(buf_v, acc_sh.at[idx2_v.at[k]], add=True)
                return 0
            lax.fori_loop(0, 50, _chunk, 0)
            plsc.subcore_barrier()

            row_out = base + zrow0

            @pl.when(c == 0)
            def _wo0():
                _writeout(sums_hbm.at[q, pl.ds(row_out, WIN // 16)])

            @pl.when(c == 1)
            def _wo1():
                _writeout(sums_hbm.at[2 + q, pl.ds(row_out, WIN // 16)])

            plsc.subcore_barrier()

        cnt_active = c == (w // 2)

        @pl.when(cnt_active)
        def _cz():
            _zero_acc()

        plsc.subcore_barrier()

        @pl.when(cnt_active)
        def _cs():
            def _cchunk(k, _):
                pltpu.sync_copy(ones_v, acc_sh.at[idx2_v.at[k]], add=True)
                return 0
            lax.fori_loop(0, 50, _cchunk, 0)

        plsc.subcore_barrier()

        @pl.when(cnt_active)
        def _cw():
            _writeout(cnt_hbm.at[pl.ds(base + zrow0, WIN // 16)])

        plsc.subcore_barrier()


def _run_scatter(h3, seg2d):
    f = pl.kernel(
        _scatter_body,
        mesh=plsc.VectorSubcoreMesh(**_SC_MESH),
        out_type=[
            jax.ShapeDtypeStruct((NCHUNK, GPAD, CCOLS), jnp.float32),
            jax.ShapeDtypeStruct((GPAD, CCOLS), jnp.float32),
        ],
        scratch_types=[
            pltpu.VMEM((56, 128), jnp.int32),
            pltpu.VMEM((56, 128), jnp.int32),
            pltpu.VMEM((128, CCOLS), jnp.float32),
            pltpu.VMEM((56, 128), jnp.float32),
            pltpu.VMEM((128, CCOLS), jnp.float32),
            pltpu.VMEM_SHARED((ACC_ROWS, 128), jnp.float32),
        ],
    )
    return f(h3, seg2d)


def _norm_body(sums_ref, cnt_ref, out_ref):
    cnt = jnp.maximum(cnt_ref[:, 0:1], 1.0)
    cat = jnp.concatenate([sums_ref[p] for p in range(NCHUNK)], axis=-1)
    out_ref[...] = cat / cnt


def _run_norm(sums, cnt):
    gblk = 128
    return pl.pallas_call(
        _norm_body,
        grid=(GPAD // gblk,),
        in_specs=[
            pl.BlockSpec((NCHUNK, gblk, CCOLS), lambda i: (0, i, 0)),
            pl.BlockSpec((gblk, CCOLS), lambda i: (i, 0)),
        ],
        out_specs=pl.BlockSpec((gblk, HID), lambda i: (i, 0)),
        out_shape=jax.ShapeDtypeStruct((GPAD, HID), jnp.float32),
    )(sums, cnt)


def _gather_body(means_hbm, seg_hbm, out_hbm, idx_v, buf_v, sem):
    c = lax.axis_index("c")
    s = lax.axis_index("s")
    w = s * 2 + c
    pltpu.sync_copy(seg_hbm.at[pl.ds(w * 32, 32)], idx_v)

    def _chunk(k, _):
        pltpu.async_copy(means_hbm.at[idx_v.at[k]], buf_v, sem).wait()
        pltpu.sync_copy(buf_v, out_hbm.at[pl.ds(w * 3200 + k * 128, 128)])
        return 0
    lax.fori_loop(0, 25, _chunk, 0)


def _run_gather(means, seg2d):
    f = pl.kernel(
        _gather_body,
        mesh=plsc.VectorSubcoreMesh(**_SC_MESH),
        out_type=jax.ShapeDtypeStruct((NPAD, HID), jnp.float32),
        scratch_types=[
            pltpu.VMEM((32, 128), jnp.int32),
            pltpu.VMEM((128, HID), jnp.float32),
            pltpu.SemaphoreType.DMA,
        ],
    )
    return f(means, seg2d)


def kernel(x, equi_index, atom_emb, chg_emb, ring_emb, mrs_emb,
           W1, b1, W2, b2, W3, b3):
    n = x.shape[0]
    xl = x.astype(jnp.int32)
    j = jnp.stack([
        xl[:, 0],
        N_ATOM + jnp.clip(xl[:, 1] + 10, 0, 20),
        N_ATOM + N_CHG + xl[:, 3],
        N_ATOM + N_CHG + N_RING + xl[:, 4],
    ], axis=1)
    pad_row = jnp.array([[0, N_ATOM, N_ATOM + N_CHG,
                          N_ATOM + N_CHG + N_RING]], jnp.int32)
    j = jnp.concatenate(
        [j, jnp.broadcast_to(pad_row, (NPAD - n, 4))], axis=0)
    jidx = j.reshape(NPAD // BLK, BLK, 4)

    seg = jnp.concatenate([
        equi_index.astype(jnp.int32),
        jnp.full((NPAD - n,), GARBAGE_GROUP, jnp.int32)])
    seg_a = jnp.pad(seg.reshape(16, 50, 128), ((0, 0), (0, 6), (0, 0)),
                    constant_values=GARBAGE_GROUP).reshape(16 * 56, 128)
    seg_c = jnp.pad(seg.reshape(32, 25, 128), ((0, 0), (0, 7), (0, 0)),
                    constant_values=GARBAGE_GROUP).reshape(32 * 32, 128)

    t1 = _make_t1(atom_emb, chg_emb, ring_emb, mrs_emb, W1)
    b1t = jnp.broadcast_to(b1[None, :], (8, HID))
    b2t = jnp.broadcast_to(b2[None, :], (8, HID))
    b3t = jnp.broadcast_to(b3[None, :], (8, HID))
    h3 = _run_mlp(jidx, t1, W2, W3, b1t, b2t, b3t)
    sums, cnt = _run_scatter(h3, seg_a)
    means = _run_norm(sums, cnt)
    out = _run_gather(means, seg_c)
    return out[:n]
